# Initial kernel scaffold; baseline (speedup 1.0000x reference)
import jax
import jax.numpy as jnp
from jax.experimental import pallas as pl


def kernel(x, table):
    raise NotImplementedError("write your pallas kernel here")


# speedup vs baseline: 1.2796x; 1.2796x over previous
import functools

import jax
import jax.numpy as jnp
from jax import lax
from jax.experimental import pallas as pl
from jax.experimental.pallas import tpu as pltpu
from jax.experimental.pallas import tpu_sc as plsc

BATCH = 16384
SEQ = 50
DIM = 128
B = BATCH * SEQ

CHUNK = 128
NBUF = 4


@functools.lru_cache(maxsize=None)
def _build():
    info = plsc.get_sparse_core_info()
    nc, ns = info.num_cores, info.num_subcores
    nw = nc * ns
    b_per_w = B // nw
    chunks = b_per_w // CHUNK
    steady = chunks - NBUF

    mesh = plsc.VectorSubcoreMesh(core_axis_name="c", subcore_axis_name="s")

    @functools.partial(
        pl.kernel,
        out_type=jax.ShapeDtypeStruct((B, DIM), jnp.float32),
        mesh=mesh,
        scratch_types=[
            pltpu.VMEM((chunks, CHUNK), jnp.int32),
            [pltpu.VMEM((CHUNK, DIM), jnp.float32)] * NBUF,
            [pltpu.SemaphoreType.DMA] * NBUF,
        ],
    )
    def emb_kernel(idx_hbm, table_hbm, out_hbm, idx_v, rows, sems):
        wid = lax.axis_index("s") * nc + lax.axis_index("c")
        pltpu.sync_copy(idx_hbm.at[pl.ds(wid * chunks, chunks)], idx_v)

        row0 = wid * b_per_w

        def start_gather(g, b):
            return pltpu.async_copy(table_hbm.at[idx_v.at[g]], rows[b], sems[b])

        for b in range(NBUF):
            start_gather(b, b)

        def body(outer):
            for b in range(NBUF):
                g = outer + b
                pltpu.make_async_copy(table_hbm.at[idx_v.at[g]], rows[b],
                                      sems[b]).wait()
                pltpu.sync_copy(rows[b],
                                out_hbm.at[pl.ds(row0 + g * CHUNK, CHUNK)])
                start_gather(g + NBUF, b)

        pl.loop(0, steady, step=NBUF)(body)

        for b in range(NBUF):
            g = steady + b
            pltpu.make_async_copy(table_hbm.at[idx_v.at[g]], rows[b],
                                  sems[b]).wait()
            pltpu.sync_copy(rows[b],
                            out_hbm.at[pl.ds(row0 + g * CHUNK, CHUNK)])

    return emb_kernel


def kernel(x, table):
    idx = x.reshape(B // CHUNK, CHUNK)
    out = _build()(idx, table)
    return out.reshape(BATCH, SEQ, DIM)
